# baseline (speedup 1.0000x reference)
import functools

import jax
import jax.numpy as jnp
from jax import lax
from jax.experimental import pallas as pl
from jax.experimental.pallas import tpu as pltpu
from jax.experimental.pallas import tpu_sc as plsc

_WEIGHT = 0.1
_NSEG = 32
_HB = 64
_RCH = 48


def _sc_p1_body(inp_hbm, ids_hbm, sums_hbm, aux_hbm, data_v, ids_v, acc_v,
                aux_v):
    cid = lax.axis_index("c")
    sid = lax.axis_index("s")
    wid = sid * 2 + cid
    b = wid // 16
    c0 = (wid % 16) * 6
    nchunk = 192 // _RCH
    groups = 384 // 16

    for j in range(6):
        c = c0 + j
        acc_v[pl.ds(0, 16)] = jnp.zeros((16,), jnp.float32)
        acc_v[pl.ds(16, 16)] = jnp.zeros((16,), jnp.float32)
        for ck in range(nchunk):
            pltpu.sync_copy(ids_hbm.at[b, pl.ds(ck * _RCH, _RCH)], ids_v)
            pltpu.sync_copy(inp_hbm.at[b, c, pl.ds(ck * 2 * _RCH, 2 * _RCH)],
                            data_v)

            def row_body(r, carry):
                for g in range(groups):
                    sl = pl.ds(g * 16, 16)
                    va = data_v[2 * r, sl]
                    vb = data_v[2 * r + 1, sl]
                    idx = ids_v[r, sl]
                    plsc.addupdate_scatter(acc_v, [idx], va + vb)
                return carry

            lax.fori_loop(0, _RCH, row_body, 0)
        pltpu.sync_copy(acc_v, sums_hbm.at[b * 96 + c])

    @pl.when(wid % 16 == 0)
    def _():
        for q in range(4):
            aux_v[pl.ds(q * 16, 16)] = jnp.zeros((16,), jnp.float32)
        twos = jnp.full((16,), 2.0, jnp.float32)

        def chunk_counts(ck, vm):
            pltpu.sync_copy(ids_hbm.at[b, pl.ds(ck * _RCH, _RCH)], ids_v)

            def row_body(r, vm_in):
                for g in range(groups):
                    idx = ids_v[r, pl.ds(g * 16, 16)]
                    plsc.addupdate_scatter(aux_v, [idx], twos)
                    vm_in = jnp.maximum(vm_in, idx)
                return vm_in

            return lax.fori_loop(0, _RCH, row_body, vm)

        vm = lax.fori_loop(0, nchunk, chunk_counts,
                           jnp.zeros((16,), jnp.int32))
        m = lax.reduce_max(vm, (0,)).astype(jnp.float32)
        mv = jnp.full((16,), m, jnp.float32)
        aux_v[pl.ds(32, 16)] = mv
        aux_v[pl.ds(48, 16)] = mv
        pltpu.sync_copy(aux_v, aux_hbm.at[b, 0])


def _p2_body(inp_ref, sp_ref, sums_ref, aux_ref, out_ref):
    b = pl.program_id(0)
    sums = sums_ref[0]
    cnt = aux_ref[b, 0:1, 0:_NSEG]
    m = jnp.max(aux_ref[:, 0, _NSEG:])
    seg_row = lax.broadcasted_iota(jnp.int32, (1, _NSEG), 1)
    scale = _WEIGHT * sums / (cnt + 1e-05)
    scale = jnp.where(seg_row.astype(jnp.float32) < m, scale, 0.0)
    seg_col = lax.broadcasted_iota(jnp.int32, (_NSEG, 1), 0)
    for r in range(_HB // 2):
        spi = sp_ref[0, pl.ds(r, 1), :].astype(jnp.int32)
        ohT = (spi == seg_col).astype(jnp.float32)
        delta = lax.dot_general(scale, ohT, (((1,), (0,)), ((), ())),
                                preferred_element_type=jnp.float32)
        out_ref[0, :, 2 * r, :] = inp_ref[0, :, 2 * r, :] + delta
        out_ref[0, :, 2 * r + 1, :] = inp_ref[0, :, 2 * r + 1, :] + delta


def kernel(inp, superpixel):
    B, C, H, W = inp.shape
    Hs = superpixel.shape[1]
    spw = jnp.repeat(superpixel, W // superpixel.shape[2], axis=2)
    ids = spw.astype(jnp.int32)

    sc_p1 = functools.partial(
        pl.kernel,
        mesh=plsc.VectorSubcoreMesh(core_axis_name="c", subcore_axis_name="s"),
        compiler_params=pltpu.CompilerParams(needs_layout_passes=False),
        out_type=[
            jax.ShapeDtypeStruct((B * C, _NSEG), jnp.float32),
            jax.ShapeDtypeStruct((B, 1, 64), jnp.float32),
        ],
        scratch_types=[
            pltpu.VMEM((2 * _RCH, W), jnp.float32),
            pltpu.VMEM((_RCH, W), jnp.int32),
            pltpu.VMEM((_NSEG,), jnp.float32),
            pltpu.VMEM((64,), jnp.float32),
        ],
    )(_sc_p1_body)
    sums2, aux = sc_p1(inp, ids)
    sums = sums2.reshape(B, C, _NSEG)

    nh = H // _HB
    hl = _HB // 2
    out = pl.pallas_call(
        _p2_body,
        grid=(B, nh),
        in_specs=[
            pl.BlockSpec((1, C, _HB, W), lambda b, h: (b, 0, h, 0)),
            pl.BlockSpec((1, hl, W), lambda b, h: (b, h, 0)),
            pl.BlockSpec((1, C, _NSEG), lambda b, h: (b, 0, 0)),
            pl.BlockSpec((B, 1, 64), lambda b, h: (0, 0, 0)),
        ],
        out_specs=pl.BlockSpec((1, C, _HB, W), lambda b, h: (b, 0, h, 0)),
        out_shape=jax.ShapeDtypeStruct((B, C, H, W), jnp.float32),
    )(inp, spw, sums, aux)
    return out


# speedup vs baseline: 3.4857x; 3.4857x over previous
import jax
import jax.numpy as jnp
from jax import lax
from jax.experimental import pallas as pl

_WEIGHT = 0.1
_NSEG = 32
_HB = 32


def _p1_body(inp_ref, sp_ref, sums_ref, counts_ref, max_ref):
    b = pl.program_id(0)
    h = pl.program_id(1)
    C = inp_ref.shape[1]
    W = inp_ref.shape[3]
    seg_col = lax.broadcasted_iota(jnp.int32, (_NSEG, 1), 0)
    twos = jnp.full((8, W), 2.0, jnp.float32)
    acc = None
    oh_sum = None
    for r in range(_HB // 2):
        spi = sp_ref[0, pl.ds(r, 1), :].astype(jnp.int32)
        ohT = (spi == seg_col).astype(jnp.float32)
        oh_sum = ohT if oh_sum is None else oh_sum + ohT
        xp = inp_ref[0, :, 2 * r, :] + inp_ref[0, :, 2 * r + 1, :]
        s = lax.dot_general(xp, ohT, (((1,), (1,)), ((), ())),
                            preferred_element_type=jnp.float32)
        acc = s if acc is None else acc + s
    cnt = lax.dot_general(twos, oh_sum, (((1,), (1,)), ((), ())),
                          preferred_element_type=jnp.float32)
    m = jnp.max(sp_ref[0])

    @pl.when(h == 0)
    def _():
        sums_ref[0] = acc
        counts_ref[0] = cnt

    @pl.when(h != 0)
    def _():
        sums_ref[0] += acc
        counts_ref[0] += cnt

    first = (b == 0) & (h == 0)

    @pl.when(first)
    def _():
        max_ref[0] = jnp.full((8, 128), m, jnp.float32)

    @pl.when(~first)
    def _():
        max_ref[0] = jnp.maximum(max_ref[0], m)


def _p2_body(inp_ref, sp_ref, sums_ref, counts_ref, max_ref, out_ref):
    sums = sums_ref[0]
    cnt = counts_ref[0][0:1, :]
    m = jnp.max(max_ref[0])
    seg_row = lax.broadcasted_iota(jnp.int32, (1, _NSEG), 1)
    scale = _WEIGHT * sums / (cnt + 1e-05)
    scale = jnp.where(seg_row.astype(jnp.float32) < m, scale, 0.0)
    seg_col = lax.broadcasted_iota(jnp.int32, (_NSEG, 1), 0)
    for r in range(_HB // 2):
        spi = sp_ref[0, pl.ds(r, 1), :].astype(jnp.int32)
        ohT = (spi == seg_col).astype(jnp.float32)
        delta = lax.dot_general(scale, ohT, (((1,), (0,)), ((), ())),
                                preferred_element_type=jnp.float32)
        out_ref[0, :, 2 * r, :] = inp_ref[0, :, 2 * r, :] + delta
        out_ref[0, :, 2 * r + 1, :] = inp_ref[0, :, 2 * r + 1, :] + delta


def kernel(inp, superpixel):
    B, C, H, W = inp.shape
    Hs = superpixel.shape[1]
    spw = jnp.repeat(superpixel, W // superpixel.shape[2], axis=2)
    nh = H // _HB
    hl = _HB // 2
    grid = (B, nh)
    sums, counts, maxv = pl.pallas_call(
        _p1_body,
        grid=grid,
        in_specs=[
            pl.BlockSpec((1, C, _HB, W), lambda b, h: (b, 0, h, 0)),
            pl.BlockSpec((1, hl, W), lambda b, h: (b, h, 0)),
        ],
        out_specs=[
            pl.BlockSpec((1, C, _NSEG), lambda b, h: (b, 0, 0)),
            pl.BlockSpec((1, 8, _NSEG), lambda b, h: (b, 0, 0)),
            pl.BlockSpec((1, 8, 128), lambda b, h: (0, 0, 0)),
        ],
        out_shape=[
            jax.ShapeDtypeStruct((B, C, _NSEG), jnp.float32),
            jax.ShapeDtypeStruct((B, 8, _NSEG), jnp.float32),
            jax.ShapeDtypeStruct((1, 8, 128), jnp.float32),
        ],
    )(inp, spw)

    out = pl.pallas_call(
        _p2_body,
        grid=grid,
        in_specs=[
            pl.BlockSpec((1, C, _HB, W), lambda b, h: (b, 0, h, 0)),
            pl.BlockSpec((1, hl, W), lambda b, h: (b, h, 0)),
            pl.BlockSpec((1, C, _NSEG), lambda b, h: (b, 0, 0)),
            pl.BlockSpec((1, 8, _NSEG), lambda b, h: (b, 0, 0)),
            pl.BlockSpec((1, 8, 128), lambda b, h: (0, 0, 0)),
        ],
        out_specs=pl.BlockSpec((1, C, _HB, W), lambda b, h: (b, 0, h, 0)),
        out_shape=jax.ShapeDtypeStruct((B, C, H, W), jnp.float32),
    )(inp, spw, sums, counts, maxv)
    return out


# speedup vs baseline: 3.7940x; 1.0884x over previous
import jax
import jax.numpy as jnp
from jax import lax
from jax.experimental import pallas as pl

_WEIGHT = 0.1
_NSEG = 32
_HB = 64


def _p1_body(inp_ref, sp_ref, sums_ref, counts_ref, max_ref):
    b = pl.program_id(0)
    h = pl.program_id(1)
    C = inp_ref.shape[1]
    W = inp_ref.shape[3]
    seg_col = lax.broadcasted_iota(jnp.int32, (_NSEG, 1), 0)
    twos = jnp.full((8, W), 2.0, jnp.float32)
    acc = None
    oh_sum = None
    for r in range(_HB // 2):
        spi = sp_ref[0, pl.ds(r, 1), :].astype(jnp.int32)
        ohT = (spi == seg_col).astype(jnp.float32)
        oh_sum = ohT if oh_sum is None else oh_sum + ohT
        xp = inp_ref[0, :, 2 * r, :] + inp_ref[0, :, 2 * r + 1, :]
        s = lax.dot_general(xp, ohT, (((1,), (1,)), ((), ())),
                            preferred_element_type=jnp.float32)
        acc = s if acc is None else acc + s
    cnt = lax.dot_general(twos, oh_sum, (((1,), (1,)), ((), ())),
                          preferred_element_type=jnp.float32)
    m = jnp.max(sp_ref[0])

    @pl.when(h == 0)
    def _():
        sums_ref[0] = acc
        counts_ref[0] = cnt

    @pl.when(h != 0)
    def _():
        sums_ref[0] += acc
        counts_ref[0] += cnt

    first = (b == 0) & (h == 0)

    @pl.when(first)
    def _():
        max_ref[0] = jnp.full((8, 128), m, jnp.float32)

    @pl.when(~first)
    def _():
        max_ref[0] = jnp.maximum(max_ref[0], m)


def _p2_body(inp_ref, sp_ref, sums_ref, counts_ref, max_ref, out_ref):
    sums = sums_ref[0]
    cnt = counts_ref[0][0:1, :]
    m = jnp.max(max_ref[0])
    seg_row = lax.broadcasted_iota(jnp.int32, (1, _NSEG), 1)
    scale = _WEIGHT * sums / (cnt + 1e-05)
    scale = jnp.where(seg_row.astype(jnp.float32) < m, scale, 0.0)
    seg_col = lax.broadcasted_iota(jnp.int32, (_NSEG, 1), 0)
    for r in range(_HB // 2):
        spi = sp_ref[0, pl.ds(r, 1), :].astype(jnp.int32)
        ohT = (spi == seg_col).astype(jnp.float32)
        delta = lax.dot_general(scale, ohT, (((1,), (0,)), ((), ())),
                                preferred_element_type=jnp.float32)
        out_ref[0, :, 2 * r, :] = inp_ref[0, :, 2 * r, :] + delta
        out_ref[0, :, 2 * r + 1, :] = inp_ref[0, :, 2 * r + 1, :] + delta


def kernel(inp, superpixel):
    B, C, H, W = inp.shape
    Hs = superpixel.shape[1]
    spw = jnp.repeat(superpixel, W // superpixel.shape[2], axis=2)
    nh = H // _HB
    hl = _HB // 2
    grid = (B, nh)
    sums, counts, maxv = pl.pallas_call(
        _p1_body,
        grid=grid,
        in_specs=[
            pl.BlockSpec((1, C, _HB, W), lambda b, h: (b, 0, h, 0)),
            pl.BlockSpec((1, hl, W), lambda b, h: (b, h, 0)),
        ],
        out_specs=[
            pl.BlockSpec((1, C, _NSEG), lambda b, h: (b, 0, 0)),
            pl.BlockSpec((1, 8, _NSEG), lambda b, h: (b, 0, 0)),
            pl.BlockSpec((1, 8, 128), lambda b, h: (0, 0, 0)),
        ],
        out_shape=[
            jax.ShapeDtypeStruct((B, C, _NSEG), jnp.float32),
            jax.ShapeDtypeStruct((B, 8, _NSEG), jnp.float32),
            jax.ShapeDtypeStruct((1, 8, 128), jnp.float32),
        ],
    )(inp, spw)

    out = pl.pallas_call(
        _p2_body,
        grid=grid,
        in_specs=[
            pl.BlockSpec((1, C, _HB, W), lambda b, h: (b, 0, h, 0)),
            pl.BlockSpec((1, hl, W), lambda b, h: (b, h, 0)),
            pl.BlockSpec((1, C, _NSEG), lambda b, h: (b, 0, 0)),
            pl.BlockSpec((1, 8, _NSEG), lambda b, h: (b, 0, 0)),
            pl.BlockSpec((1, 8, 128), lambda b, h: (0, 0, 0)),
        ],
        out_specs=pl.BlockSpec((1, C, _HB, W), lambda b, h: (b, 0, h, 0)),
        out_shape=jax.ShapeDtypeStruct((B, C, H, W), jnp.float32),
    )(inp, spw, sums, counts, maxv)
    return out
